# baseline (speedup 1.0000x reference)
import functools

import jax
import jax.numpy as jnp
import numpy as np
from jax import lax
from jax.experimental import pallas as pl
from jax.experimental.pallas import tpu as pltpu
from jax.experimental.pallas import tpu_sc as plsc

IN_DIM = 8192
OUT_DIM = 16384
BATCH = 1024

_COEFF = np.array(
    [
        [0, 0, 0, 0],
        [0, 0, 0, 1],
        [0, 1, 0, -1],
        [0, 1, 0, 0],
        [0, 0, 1, -1],
        [0, 0, 1, 0],
        [0, 1, 1, -2],
        [0, 1, 1, -1],
        [1, -1, -1, 1],
        [1, -1, -1, 2],
        [1, 0, -1, 0],
        [1, 0, -1, 1],
        [1, -1, 0, 0],
        [1, -1, 0, 1],
        [1, 0, 0, -1],
        [1, 0, 0, 0],
    ],
    dtype=np.float32,
)

_CG = 2048


def _meta_body(ct_ref, lt_ref, a_ref, b_ref, m_ref):
    l = lt_ref[...]
    m = jnp.max(l, axis=0, keepdims=True)
    e = jnp.exp(l - m)
    s = jnp.sum(e, axis=0, keepdims=True)
    p = e / s
    w = jnp.dot(ct_ref[...], p, preferred_element_type=jnp.float32)
    wi = lax.bitcast_convert_type(w, jnp.int32)
    a = a_ref[0]
    b = b_ref[0]
    a_t = ((a & -128) << 3) + (a & 127)
    b_t = ((b & -128) << 3) + (b & 127)
    pad = jnp.zeros((2, a.shape[1]), jnp.int32)
    m_ref[...] = jnp.concatenate([a_t, b_t, wi, pad], axis=0)


def _pack_meta(logits, a_idx, b_idx):
    lt = logits.T
    ct = jnp.asarray(_COEFF.T)
    a2 = a_idx.reshape(_NCHUNK, 1, _G)
    b2 = b_idx.reshape(_NCHUNK, 1, _G)
    meta = pl.pallas_call(
        _meta_body,
        grid=(_NCHUNK,),
        in_specs=[
            pl.BlockSpec((4, 16), lambda i: (0, 0)),
            pl.BlockSpec((16, _G), lambda i: (0, i)),
            pl.BlockSpec((1, 1, _G), lambda i: (i, 0, 0)),
            pl.BlockSpec((1, 1, _G), lambda i: (i, 0, 0)),
        ],
        out_specs=pl.BlockSpec((8, _G), lambda i: (i, 0)),
        out_shape=jax.ShapeDtypeStruct((8 * _NCHUNK, _G), jnp.int32),
    )(ct, lt, a2, b2)
    m4 = meta.reshape(_NCHUNK, 8, _G // 128, 128)
    return m4.transpose(0, 2, 1, 3).reshape(-1)


_NC, _NS = 2, 16
_NW = _NC * _NS
_RPT = BATCH // _NW
_NB = 8
_NPASS = _RPT // _NB
_G = 1024
_NCHUNK = OUT_DIM // _G
_MG = 8 * _G


def _sc_body(x_hbm, meta_hbm, y_hbm, x_l, m_l0, m_l1, y_l0, y_l1, m_sh,
             si0, si1, so0, so1):
    c = lax.axis_index("c")
    s = lax.axis_index("s")
    wid = s * _NC + c
    base = wid * _RPT
    m_l = (m_l0, m_l1)
    y_l = (y_l0, y_l1)
    si = (si0, si1)
    so = (so0, so1)

    stripe = 8 * OUT_DIM // _NS
    pltpu.sync_copy(
        meta_hbm.at[pl.ds(s * stripe, stripe)], m_sh.at[pl.ds(s * stripe, stripe)]
    )
    plsc.subcore_barrier()

    def issue_in(ci, sl):
        pltpu.async_copy(m_sh.at[pl.ds(ci * _MG, _MG)], m_l[sl], si[sl])

    def wait_in(sl):
        pltpu.make_async_copy(m_sh.at[pl.ds(0, _MG)], m_l[sl], si[sl]).wait()

    def issue_out(row0, ci, sl):
        pltpu.async_copy(
            y_l[sl],
            y_hbm.at[pl.ds(row0 * OUT_DIM + 8 * ci * _G, _NB * _G)],
            so[sl],
        )

    def wait_out(sl):
        pltpu.make_async_copy(
            y_l[sl], y_hbm.at[pl.ds(0, _NB * _G)], so[sl]
        ).wait()

    def compute(sl):
        ml = m_l[sl]
        yl = y_l[sl]

        @plsc.parallel_loop(0, _G // 16, unroll=2)
        def _(gi):
            off = gi * 16
            soff = ((off >> 7) << 10) + (off & 127)
            av = ml[pl.ds(soff, 16)]
            bv = ml[pl.ds(soff + 128, 16)]
            w0 = plsc.bitcast(ml[pl.ds(soff + 2 * 128, 16)], jnp.float32)
            wa = plsc.bitcast(ml[pl.ds(soff + 3 * 128, 16)], jnp.float32)
            wb = plsc.bitcast(ml[pl.ds(soff + 4 * 128, 16)], jnp.float32)
            wab = plsc.bitcast(ml[pl.ds(soff + 5 * 128, 16)], jnp.float32)
            for n in range(_NB):
                xs = x_l.at[pl.ds(n * 128, _NB * IN_DIM - n * 128)]
                a = plsc.load_gather(xs, [av])
                b = plsc.load_gather(xs, [bv])
                yl[pl.ds(soff + n * 128, 16)] = w0 + wa * a + wb * b + wab * (a * b)

    for p in range(_NPASS):
        row0 = base + p * _NB
        issue_in(0, 0)
        pltpu.sync_copy(x_hbm.at[pl.ds(row0 * IN_DIM, _NB * IN_DIM)], x_l)

        @pl.loop(0, _NCHUNK // 2)
        def _(k):
            ci0 = k * 2
            ci1 = ci0 + 1
            issue_in(ci1, 1)
            wait_in(0)
            if p == 0:
                @pl.when(k > 0)
                def _():
                    wait_out(0)
            else:
                wait_out(0)
            compute(0)
            issue_out(row0, ci0, 0)

            @pl.when(k < _NCHUNK // 2 - 1)
            def _():
                issue_in(ci0 + 2, 0)

            wait_in(1)
            if p == 0:
                @pl.when(k > 0)
                def _():
                    wait_out(1)
            else:
                wait_out(1)
            compute(1)
            issue_out(row0, ci1, 1)

    wait_out(0)
    wait_out(1)


@functools.partial(jax.jit, donate_argnums=())
def _sc_main(x_flat, meta_flat):
    mesh = plsc.VectorSubcoreMesh(
        core_axis_name="c", subcore_axis_name="s", num_cores=_NC, num_subcores=_NS
    )
    fn = pl.kernel(
        _sc_body,
        out_type=jax.ShapeDtypeStruct((BATCH * OUT_DIM,), jnp.float32),
        mesh=mesh,
        scratch_types=[
            pltpu.VMEM((_NB * IN_DIM,), jnp.float32),
            pltpu.VMEM((_MG,), jnp.int32),
            pltpu.VMEM((_MG,), jnp.int32),
            pltpu.VMEM((_NB * _G,), jnp.float32),
            pltpu.VMEM((_NB * _G,), jnp.float32),
            pltpu.VMEM_SHARED((8 * OUT_DIM,), jnp.int32),
            pltpu.SemaphoreType.DMA,
            pltpu.SemaphoreType.DMA,
            pltpu.SemaphoreType.DMA,
            pltpu.SemaphoreType.DMA,
        ],
        compiler_params=pltpu.CompilerParams(needs_layout_passes=False),
    )
    y = fn(x_flat, meta_flat)
    y4 = y.reshape(BATCH // 8, OUT_DIM // 128, 8, 128)
    return y4.transpose(0, 2, 1, 3).reshape(BATCH, OUT_DIM)


def kernel(x, logits, a_idx, b_idx):
    meta = _pack_meta(logits, a_idx, b_idx)
    x_tiled = x.reshape(BATCH // 8, 8, IN_DIM // 128, 128)
    x_tiled = x_tiled.transpose(0, 2, 1, 3).reshape(-1)
    return _sc_main(x_tiled, meta)


# speedup vs baseline: 1.2364x; 1.2364x over previous
import functools

import jax
import jax.numpy as jnp
import numpy as np
from jax import lax
from jax.experimental import pallas as pl
from jax.experimental.pallas import tpu as pltpu
from jax.experimental.pallas import tpu_sc as plsc

IN_DIM = 8192
OUT_DIM = 16384
BATCH = 1024

_COEFF = np.array(
    [
        [0, 0, 0, 0],
        [0, 0, 0, 1],
        [0, 1, 0, -1],
        [0, 1, 0, 0],
        [0, 0, 1, -1],
        [0, 0, 1, 0],
        [0, 1, 1, -2],
        [0, 1, 1, -1],
        [1, -1, -1, 1],
        [1, -1, -1, 2],
        [1, 0, -1, 0],
        [1, 0, -1, 1],
        [1, -1, 0, 0],
        [1, -1, 0, 1],
        [1, 0, 0, -1],
        [1, 0, 0, 0],
    ],
    dtype=np.float32,
)

_CG = 2048


def _coeff_body(ct_ref, lt_ref, w_ref):
    l = lt_ref[...]
    m = jnp.max(l, axis=0, keepdims=True)
    e = jnp.exp(l - m)
    s = jnp.sum(e, axis=0, keepdims=True)
    p = e / s
    w_ref[...] = jnp.dot(ct_ref[...], p, preferred_element_type=jnp.float32)


def _coefficients(logits):
    lt = logits.T
    ct = jnp.asarray(_COEFF.T)
    return pl.pallas_call(
        _coeff_body,
        grid=(OUT_DIM // _CG,),
        in_specs=[
            pl.BlockSpec((4, 16), lambda i: (0, 0)),
            pl.BlockSpec((16, _CG), lambda i: (0, i)),
        ],
        out_specs=pl.BlockSpec((4, _CG), lambda i: (0, i)),
        out_shape=jax.ShapeDtypeStruct((4, OUT_DIM), jnp.float32),
    )(ct, lt)


_NC, _NS = 2, 16
_NW = _NC * _NS
_RPT = BATCH // _NW
_NB = 8
_NPASS = _RPT // _NB
_G = 2048
_NCHUNK = OUT_DIM // _G
_MG = 6 * _G


def _sc_body(x_hbm, meta_hbm, y_hbm, x_l, m_l0, m_l1, y_l0, y_l1, m_sh,
             si0, si1, so0, so1):
    c = lax.axis_index("c")
    s = lax.axis_index("s")
    wid = s * _NC + c
    base = wid * _RPT
    m_l = (m_l0, m_l1)
    y_l = (y_l0, y_l1)
    si = (si0, si1)
    so = (so0, so1)

    stripe = 6 * OUT_DIM // _NS
    pltpu.sync_copy(
        meta_hbm.at[pl.ds(s * stripe, stripe)], m_sh.at[pl.ds(s * stripe, stripe)]
    )
    plsc.subcore_barrier()

    def issue_in(ci, sl):
        pltpu.async_copy(m_sh.at[pl.ds(ci * _MG, _MG)], m_l[sl], si[sl])

    def wait_in(sl):
        pltpu.make_async_copy(m_sh.at[pl.ds(0, _MG)], m_l[sl], si[sl]).wait()

    def issue_out(row0, ci, sl):
        pltpu.async_copy(
            y_l[sl],
            y_hbm.at[pl.ds(row0 * OUT_DIM + 8 * ci * _G, _NB * _G)],
            so[sl],
        )

    def wait_out(sl):
        pltpu.make_async_copy(
            y_l[sl], y_hbm.at[pl.ds(0, _NB * _G)], so[sl]
        ).wait()

    def compute(sl):
        ml = m_l[sl]
        yl = y_l[sl]

        @plsc.parallel_loop(0, _G // 16, unroll=2)
        def _(gi):
            off = gi * 16
            av = ml[pl.ds(off, 16)]
            bv = ml[pl.ds(_G + off, 16)]
            w0 = plsc.bitcast(ml[pl.ds(2 * _G + off, 16)], jnp.float32)
            wa = plsc.bitcast(ml[pl.ds(3 * _G + off, 16)], jnp.float32)
            wb = plsc.bitcast(ml[pl.ds(4 * _G + off, 16)], jnp.float32)
            wab = plsc.bitcast(ml[pl.ds(5 * _G + off, 16)], jnp.float32)
            soff = ((off >> 7) << 10) + (off & 127)
            for n in range(_NB):
                xs = x_l.at[pl.ds(n * 128, _NB * IN_DIM - n * 128)]
                a = plsc.load_gather(xs, [av])
                b = plsc.load_gather(xs, [bv])
                yl[pl.ds(soff + n * 128, 16)] = w0 + wa * a + wb * b + wab * (a * b)

    for p in range(_NPASS):
        row0 = base + p * _NB
        issue_in(0, 0)
        pltpu.sync_copy(x_hbm.at[pl.ds(row0 * IN_DIM, _NB * IN_DIM)], x_l)

        @pl.loop(0, _NCHUNK // 2)
        def _(k):
            ci0 = k * 2
            ci1 = ci0 + 1
            issue_in(ci1, 1)
            wait_in(0)
            if p == 0:
                @pl.when(k > 0)
                def _():
                    wait_out(0)
            else:
                wait_out(0)
            compute(0)
            issue_out(row0, ci0, 0)

            @pl.when(k < _NCHUNK // 2 - 1)
            def _():
                issue_in(ci0 + 2, 0)

            wait_in(1)
            if p == 0:
                @pl.when(k > 0)
                def _():
                    wait_out(1)
            else:
                wait_out(1)
            compute(1)
            issue_out(row0, ci1, 1)

    wait_out(0)
    wait_out(1)


@functools.partial(jax.jit, donate_argnums=())
def _sc_main(x_flat, meta_flat):
    mesh = plsc.VectorSubcoreMesh(
        core_axis_name="c", subcore_axis_name="s", num_cores=_NC, num_subcores=_NS
    )
    fn = pl.kernel(
        _sc_body,
        out_type=jax.ShapeDtypeStruct((BATCH * OUT_DIM,), jnp.float32),
        mesh=mesh,
        scratch_types=[
            pltpu.VMEM((_NB * IN_DIM,), jnp.float32),
            pltpu.VMEM((_MG,), jnp.int32),
            pltpu.VMEM((_MG,), jnp.int32),
            pltpu.VMEM((_NB * _G,), jnp.float32),
            pltpu.VMEM((_NB * _G,), jnp.float32),
            pltpu.VMEM_SHARED((6 * OUT_DIM,), jnp.int32),
            pltpu.SemaphoreType.DMA,
            pltpu.SemaphoreType.DMA,
            pltpu.SemaphoreType.DMA,
            pltpu.SemaphoreType.DMA,
        ],
        compiler_params=pltpu.CompilerParams(needs_layout_passes=False),
    )
    y = fn(x_flat, meta_flat)
    y4 = y.reshape(BATCH // 8, OUT_DIM // 128, 8, 128)
    return y4.transpose(0, 2, 1, 3).reshape(BATCH, OUT_DIM)


def kernel(x, logits, a_idx, b_idx):
    w = _coefficients(logits)
    wi = lax.bitcast_convert_type(w, jnp.int32)
    a_t = ((a_idx & -128) << 3) + (a_idx & 127)
    b_t = ((b_idx & -128) << 3) + (b_idx & 127)
    a2 = a_t.reshape(_NCHUNK, 1, _G)
    b2 = b_t.reshape(_NCHUNK, 1, _G)
    wi3 = wi.reshape(4, _NCHUNK, _G).transpose(1, 0, 2)
    meta = jnp.concatenate([a2, b2, wi3], axis=1).reshape(-1)
    x_tiled = x.reshape(BATCH // 8, 8, IN_DIM // 128, 128)
    x_tiled = x_tiled.transpose(0, 2, 1, 3).reshape(-1)
    return _sc_main(x_tiled, meta)


# speedup vs baseline: 1.2612x; 1.0200x over previous
import functools

import jax
import jax.numpy as jnp
import numpy as np
from jax import lax
from jax.experimental import pallas as pl
from jax.experimental.pallas import tpu as pltpu
from jax.experimental.pallas import tpu_sc as plsc

IN_DIM = 8192
OUT_DIM = 16384
BATCH = 1024

_COEFF = np.array(
    [
        [0, 0, 0, 0],
        [0, 0, 0, 1],
        [0, 1, 0, -1],
        [0, 1, 0, 0],
        [0, 0, 1, -1],
        [0, 0, 1, 0],
        [0, 1, 1, -2],
        [0, 1, 1, -1],
        [1, -1, -1, 1],
        [1, -1, -1, 2],
        [1, 0, -1, 0],
        [1, 0, -1, 1],
        [1, -1, 0, 0],
        [1, -1, 0, 1],
        [1, 0, 0, -1],
        [1, 0, 0, 0],
    ],
    dtype=np.float32,
)

_CG = 2048


def _coeff_body(ct_ref, lt_ref, w_ref):
    l = lt_ref[...]
    m = jnp.max(l, axis=0, keepdims=True)
    e = jnp.exp(l - m)
    s = jnp.sum(e, axis=0, keepdims=True)
    p = e / s
    w_ref[...] = jnp.dot(ct_ref[...], p, preferred_element_type=jnp.float32)


def _coefficients(logits):
    lt = logits.T
    ct = jnp.asarray(_COEFF.T)
    return pl.pallas_call(
        _coeff_body,
        grid=(OUT_DIM // _CG,),
        in_specs=[
            pl.BlockSpec((4, 16), lambda i: (0, 0)),
            pl.BlockSpec((16, _CG), lambda i: (0, i)),
        ],
        out_specs=pl.BlockSpec((4, _CG), lambda i: (0, i)),
        out_shape=jax.ShapeDtypeStruct((4, OUT_DIM), jnp.float32),
    )(ct, lt)


_NC, _NS = 2, 16
_NW = _NC * _NS
_RPT = BATCH // _NW
_NB = 8
_NPASS = _RPT // _NB
_G = 2048
_NCHUNK = OUT_DIM // _G
_MG = 5 * _G


def _sc_body(x_hbm, meta_hbm, y_hbm, x_l, m_l0, m_l1, y_l0, y_l1, m_sh,
             si0, si1, so0, so1):
    c = lax.axis_index("c")
    s = lax.axis_index("s")
    wid = s * _NC + c
    base = wid * _RPT
    m_l = (m_l0, m_l1)
    y_l = (y_l0, y_l1)
    si = (si0, si1)
    so = (so0, so1)

    stripe = 5 * OUT_DIM // _NS
    pltpu.sync_copy(
        meta_hbm.at[pl.ds(s * stripe, stripe)], m_sh.at[pl.ds(s * stripe, stripe)]
    )
    plsc.subcore_barrier()

    def issue_in(ci, sl):
        pltpu.async_copy(m_sh.at[pl.ds(ci * _MG, _MG)], m_l[sl], si[sl])

    def wait_in(sl):
        pltpu.make_async_copy(m_sh.at[pl.ds(0, _MG)], m_l[sl], si[sl]).wait()

    def issue_out(row0, ci, sl):
        pltpu.async_copy(
            y_l[sl],
            y_hbm.at[pl.ds(row0 * OUT_DIM + 8 * ci * _G, _NB * _G)],
            so[sl],
        )

    def wait_out(sl):
        pltpu.make_async_copy(
            y_l[sl], y_hbm.at[pl.ds(0, _NB * _G)], so[sl]
        ).wait()

    def compute(sl):
        ml = m_l[sl]
        yl = y_l[sl]

        @plsc.parallel_loop(0, _G // 16, unroll=2)
        def _(gi):
            off = gi * 16
            avb = ml[pl.ds(off, 16)]
            av = avb & 0xFFFF
            bv = lax.shift_right_logical(avb, 16)
            w0 = plsc.bitcast(ml[pl.ds(1 * _G + off, 16)], jnp.float32)
            wa = plsc.bitcast(ml[pl.ds(2 * _G + off, 16)], jnp.float32)
            wb = plsc.bitcast(ml[pl.ds(3 * _G + off, 16)], jnp.float32)
            wab = plsc.bitcast(ml[pl.ds(4 * _G + off, 16)], jnp.float32)
            soff = ((off >> 7) << 10) + (off & 127)
            for n in range(_NB):
                xs = x_l.at[pl.ds(n * 128, _NB * IN_DIM - n * 128)]
                a = plsc.load_gather(xs, [av])
                b = plsc.load_gather(xs, [bv])
                yl[pl.ds(soff + n * 128, 16)] = w0 + wa * a + wb * b + wab * (a * b)

    for p in range(_NPASS):
        row0 = base + p * _NB
        issue_in(0, 0)
        pltpu.sync_copy(x_hbm.at[pl.ds(row0 * IN_DIM, _NB * IN_DIM)], x_l)

        @pl.loop(0, _NCHUNK // 2)
        def _(k):
            ci0 = k * 2
            ci1 = ci0 + 1
            issue_in(ci1, 1)
            wait_in(0)
            if p == 0:
                @pl.when(k > 0)
                def _():
                    wait_out(0)
            else:
                wait_out(0)
            compute(0)
            issue_out(row0, ci0, 0)

            @pl.when(k < _NCHUNK // 2 - 1)
            def _():
                issue_in(ci0 + 2, 0)

            wait_in(1)
            if p == 0:
                @pl.when(k > 0)
                def _():
                    wait_out(1)
            else:
                wait_out(1)
            compute(1)
            issue_out(row0, ci1, 1)

    wait_out(0)
    wait_out(1)


@functools.partial(jax.jit, donate_argnums=())
def _sc_main(x_flat, meta_flat):
    mesh = plsc.VectorSubcoreMesh(
        core_axis_name="c", subcore_axis_name="s", num_cores=_NC, num_subcores=_NS
    )
    fn = pl.kernel(
        _sc_body,
        out_type=jax.ShapeDtypeStruct((BATCH * OUT_DIM,), jnp.float32),
        mesh=mesh,
        scratch_types=[
            pltpu.VMEM((_NB * IN_DIM,), jnp.float32),
            pltpu.VMEM((_MG,), jnp.int32),
            pltpu.VMEM((_MG,), jnp.int32),
            pltpu.VMEM((_NB * _G,), jnp.float32),
            pltpu.VMEM((_NB * _G,), jnp.float32),
            pltpu.VMEM_SHARED((5 * OUT_DIM,), jnp.int32),
            pltpu.SemaphoreType.DMA,
            pltpu.SemaphoreType.DMA,
            pltpu.SemaphoreType.DMA,
            pltpu.SemaphoreType.DMA,
        ],
        compiler_params=pltpu.CompilerParams(needs_layout_passes=False),
    )
    y = fn(x_flat, meta_flat)
    y4 = y.reshape(BATCH // 8, OUT_DIM // 128, 8, 128)
    return y4.transpose(0, 2, 1, 3).reshape(BATCH, OUT_DIM)


def kernel(x, logits, a_idx, b_idx):
    w = _coefficients(logits)
    wi = lax.bitcast_convert_type(w, jnp.int32)
    a_t = ((a_idx & -128) << 3) + (a_idx & 127)
    b_t = ((b_idx & -128) << 3) + (b_idx & 127)
    ab = a_t | (b_t << 16)
    ab2 = ab.reshape(_NCHUNK, 1, _G)
    wi3 = wi.reshape(4, _NCHUNK, _G).transpose(1, 0, 2)
    meta = jnp.concatenate([ab2, wi3], axis=1).reshape(-1)
    x_tiled = x.reshape(BATCH // 8, 8, IN_DIM // 128, 128)
    x_tiled = x_tiled.transpose(0, 2, 1, 3).reshape(-1)
    return _sc_main(x_tiled, meta)


# speedup vs baseline: 1.2691x; 1.0063x over previous
import functools

import jax
import jax.numpy as jnp
import numpy as np
from jax import lax
from jax.experimental import pallas as pl
from jax.experimental.pallas import tpu as pltpu
from jax.experimental.pallas import tpu_sc as plsc

IN_DIM = 8192
OUT_DIM = 16384
BATCH = 1024

_COEFF = np.array(
    [
        [0, 0, 0, 0],
        [0, 0, 0, 1],
        [0, 1, 0, -1],
        [0, 1, 0, 0],
        [0, 0, 1, -1],
        [0, 0, 1, 0],
        [0, 1, 1, -2],
        [0, 1, 1, -1],
        [1, -1, -1, 1],
        [1, -1, -1, 2],
        [1, 0, -1, 0],
        [1, 0, -1, 1],
        [1, -1, 0, 0],
        [1, -1, 0, 1],
        [1, 0, 0, -1],
        [1, 0, 0, 0],
    ],
    dtype=np.float32,
)

_CG = 2048


def _coeff_body(ct_ref, lt_ref, w_ref):
    l = lt_ref[...]
    m = jnp.max(l, axis=0, keepdims=True)
    e = jnp.exp(l - m)
    s = jnp.sum(e, axis=0, keepdims=True)
    p = e / s
    w_ref[...] = jnp.dot(ct_ref[...], p, preferred_element_type=jnp.float32)


def _coefficients(logits):
    lt = logits.T
    ct = jnp.asarray(_COEFF.T)
    return pl.pallas_call(
        _coeff_body,
        grid=(OUT_DIM // _CG,),
        in_specs=[
            pl.BlockSpec((4, 16), lambda i: (0, 0)),
            pl.BlockSpec((16, _CG), lambda i: (0, i)),
        ],
        out_specs=pl.BlockSpec((4, _CG), lambda i: (0, i)),
        out_shape=jax.ShapeDtypeStruct((4, OUT_DIM), jnp.float32),
    )(ct, lt)


_NC, _NS = 2, 16
_NW = _NC * _NS
_RPT = BATCH // _NW
_NB = 8
_NPASS = _RPT // _NB
_G = 2048
_NCHUNK = OUT_DIM // _G
_MG = 5 * _G


def _sc_body(x_hbm, meta_hbm, y_hbm, x_l, m_l0, m_l1, y_l0, y_l1, m_sh,
             si0, si1, so0, so1, sx):
    c = lax.axis_index("c")
    s = lax.axis_index("s")
    wid = s * _NC + c
    base = wid * _RPT
    m_l = (m_l0, m_l1)
    y_l = (y_l0, y_l1)
    si = (si0, si1)
    so = (so0, so1)

    def issue_x(p):
        row0 = base + p * _NB
        pltpu.async_copy(x_hbm.at[pl.ds(row0 * IN_DIM, _NB * IN_DIM)], x_l, sx)

    def wait_x():
        pltpu.make_async_copy(x_hbm.at[pl.ds(0, _NB * IN_DIM)], x_l, sx).wait()

    issue_x(0)

    stripe = 5 * OUT_DIM // _NS
    pltpu.sync_copy(
        meta_hbm.at[pl.ds(s * stripe, stripe)], m_sh.at[pl.ds(s * stripe, stripe)]
    )
    plsc.subcore_barrier()

    def issue_in(ci, sl):
        pltpu.async_copy(m_sh.at[pl.ds(ci * _MG, _MG)], m_l[sl], si[sl])

    def wait_in(sl):
        pltpu.make_async_copy(m_sh.at[pl.ds(0, _MG)], m_l[sl], si[sl]).wait()

    def issue_out(row0, ci, sl):
        pltpu.async_copy(
            y_l[sl],
            y_hbm.at[pl.ds(row0 * OUT_DIM + 8 * ci * _G, _NB * _G)],
            so[sl],
        )

    def wait_out(sl):
        pltpu.make_async_copy(
            y_l[sl], y_hbm.at[pl.ds(0, _NB * _G)], so[sl]
        ).wait()

    def compute(sl):
        ml = m_l[sl]
        yl = y_l[sl]

        @plsc.parallel_loop(0, _G // 16, unroll=2)
        def _(gi):
            off = gi * 16
            avb = ml[pl.ds(off, 16)]
            av = avb & 0xFFFF
            bv = lax.shift_right_logical(avb, 16)
            w0 = plsc.bitcast(ml[pl.ds(1 * _G + off, 16)], jnp.float32)
            wa = plsc.bitcast(ml[pl.ds(2 * _G + off, 16)], jnp.float32)
            wb = plsc.bitcast(ml[pl.ds(3 * _G + off, 16)], jnp.float32)
            wab = plsc.bitcast(ml[pl.ds(4 * _G + off, 16)], jnp.float32)
            soff = ((off >> 7) << 10) + (off & 127)
            for n in range(_NB):
                xs = x_l.at[pl.ds(n * 128, _NB * IN_DIM - n * 128)]
                a = plsc.load_gather(xs, [av])
                b = plsc.load_gather(xs, [bv])
                yl[pl.ds(soff + n * 128, 16)] = w0 + wa * a + wb * b + wab * (a * b)

    for p in range(_NPASS):
        row0 = base + p * _NB
        if p > 0:
            issue_x(p)
        issue_in(0, 0)
        wait_x()

        @pl.loop(0, _NCHUNK // 2)
        def _(k):
            ci0 = k * 2
            ci1 = ci0 + 1
            issue_in(ci1, 1)
            wait_in(0)
            if p == 0:
                @pl.when(k > 0)
                def _():
                    wait_out(0)
            else:
                wait_out(0)
            compute(0)
            issue_out(row0, ci0, 0)

            @pl.when(k < _NCHUNK // 2 - 1)
            def _():
                issue_in(ci0 + 2, 0)

            wait_in(1)
            if p == 0:
                @pl.when(k > 0)
                def _():
                    wait_out(1)
            else:
                wait_out(1)
            compute(1)
            issue_out(row0, ci1, 1)

    wait_out(0)
    wait_out(1)


@functools.partial(jax.jit, donate_argnums=())
def _sc_main(x_flat, meta_flat):
    mesh = plsc.VectorSubcoreMesh(
        core_axis_name="c", subcore_axis_name="s", num_cores=_NC, num_subcores=_NS
    )
    fn = pl.kernel(
        _sc_body,
        out_type=jax.ShapeDtypeStruct((BATCH * OUT_DIM,), jnp.float32),
        mesh=mesh,
        scratch_types=[
            pltpu.VMEM((_NB * IN_DIM,), jnp.float32),
            pltpu.VMEM((_MG,), jnp.int32),
            pltpu.VMEM((_MG,), jnp.int32),
            pltpu.VMEM((_NB * _G,), jnp.float32),
            pltpu.VMEM((_NB * _G,), jnp.float32),
            pltpu.VMEM_SHARED((5 * OUT_DIM,), jnp.int32),
            pltpu.SemaphoreType.DMA,
            pltpu.SemaphoreType.DMA,
            pltpu.SemaphoreType.DMA,
            pltpu.SemaphoreType.DMA,
            pltpu.SemaphoreType.DMA,
        ],
        compiler_params=pltpu.CompilerParams(needs_layout_passes=False),
    )
    y = fn(x_flat, meta_flat)
    y4 = y.reshape(BATCH // 8, OUT_DIM // 128, 8, 128)
    return y4.transpose(0, 2, 1, 3).reshape(BATCH, OUT_DIM)


def kernel(x, logits, a_idx, b_idx):
    w = _coefficients(logits)
    wi = lax.bitcast_convert_type(w, jnp.int32)
    a_t = ((a_idx & -128) << 3) + (a_idx & 127)
    b_t = ((b_idx & -128) << 3) + (b_idx & 127)
    ab = a_t | (b_t << 16)
    ab2 = ab.reshape(_NCHUNK, 1, _G)
    wi3 = wi.reshape(4, _NCHUNK, _G).transpose(1, 0, 2)
    meta = jnp.concatenate([ab2, wi3], axis=1).reshape(-1)
    x_tiled = x.reshape(BATCH // 8, 8, IN_DIM // 128, 128)
    x_tiled = x_tiled.transpose(0, 2, 1, 3).reshape(-1)
    return _sc_main(x_tiled, meta)
